# baseline (speedup 1.0000x reference)
import jax
import jax.numpy as jnp
from jax.experimental import pallas as pl
from jax.experimental.pallas import tpu as pltpu

B = 8192
D = 384
H = 64
R = 8
NL = 64
TEMP = 1.0
AUX = 0.05

BLK = 512
NBLK = B // BLK

_HI = jax.lax.Precision.HIGHEST


def _v_kernel(llm_ref, vwt_ref, vb_ref, out_ref):
    vm = jnp.dot(llm_ref[:], vwt_ref[:], preferred_element_type=jnp.float32,
                 precision=_HI) + vb_ref[:]
    for r in range(R):
        sl = slice(r * H, (r + 1) * H)
        vr = vm[:, sl]
        n = jnp.sqrt(jnp.sum(vr * vr, axis=1, keepdims=True))
        out_ref[:, sl] = vr / jnp.maximum(n, 1e-12)


def _aux_kernel(psum_ref, msum_ref, aux_ref):
    p = jnp.sum(psum_ref[:], axis=0, keepdims=True)
    m = jnp.sum(msum_ref[:], axis=0, keepdims=True)
    aux_ref[:] = jnp.sum(p * m, axis=(0, 1), keepdims=True) * (R * AUX / (B * B))


def _main_kernel(xp_ref, xr_ref, uwg_ref, ubg_ref, v_ref,
                 rand_ref, sel_ref, logp_ref, psum_ref, msum_ref):
    xp = xp_ref[:]
    xr = xr_ref[:]
    uwg = uwg_ref[:]

    big = (jnp.dot(xp, uwg[:D], preferred_element_type=jnp.float32, precision=_HI)
           + jnp.dot(xr, uwg[D:], preferred_element_type=jnp.float32, precision=_HI)
           + ubg_ref[:])
    u_all = big[:, :R * H]
    logits = big[:, R * H:]

    m = jnp.max(logits, axis=1, keepdims=True)
    e = jnp.exp(logits - m)
    probs = e / jnp.sum(e, axis=1, keepdims=True)

    iota8 = jax.lax.broadcasted_iota(jnp.int32, (BLK, R), 1)
    m1 = jnp.max(logits, axis=1, keepdims=True)
    i1 = jnp.min(jnp.where(logits == m1, iota8, R), axis=1, keepdims=True)
    neg = jnp.where(iota8 == i1, -jnp.inf, logits)
    m2 = jnp.max(neg, axis=1, keepdims=True)
    i2 = jnp.min(jnp.where(neg == m2, iota8, R), axis=1, keepdims=True)

    ew = jnp.exp(m2 - m1)
    denom = 1.0 + ew
    w1 = 1.0 / denom
    w2 = ew / denom
    maskvec = (jnp.where(iota8 == i1, 1.0, 0.0) + jnp.where(iota8 == i2, 1.0, 0.0))

    psum_ref[:] = jnp.sum(probs, axis=0, keepdims=True).reshape(1, 1, R)
    msum_ref[:] = jnp.sum(maskvec, axis=0, keepdims=True).reshape(1, 1, R)

    chunkid = jax.lax.broadcasted_iota(jnp.int32, (BLK, R * H), 1) // H
    v = v_ref[:]
    llm_probs = jnp.zeros((BLK, NL), dtype=jnp.float32)
    for ik, wk in ((i1, w1), (i2, w2)):
        um = jnp.where(chunkid == ik, u_all, 0.0)
        nk = jnp.sqrt(jnp.sum(um * um, axis=1, keepdims=True))
        sck = jax.lax.dot_general(um, v, (((1,), (1,)), ((), ())),
                                  preferred_element_type=jnp.float32,
                                  precision=_HI) / jnp.maximum(nk, 1e-12)
        ms = jnp.max(sck, axis=1, keepdims=True)
        es = jnp.exp(sck - ms)
        ro = es / jnp.sum(es, axis=1, keepdims=True)
        llm_probs += wk * ro

    io_r = jax.lax.broadcasted_iota(jnp.int32, (NL, NL), 0)
    io_c = jax.lax.broadcasted_iota(jnp.int32, (NL, NL), 1)
    tri = (io_r <= io_c).astype(jnp.float32)
    csum = jnp.dot(llm_probs, tri, preferred_element_type=jnp.float32,
                   precision=_HI)
    iota64 = jax.lax.broadcasted_iota(jnp.int32, (BLK, NL), 1)
    gt = csum > rand_ref[:]
    idx = jnp.min(jnp.where(gt, iota64, NL), axis=1, keepdims=True)
    idx = jnp.where(idx == NL, 0, idx)
    sel_ref[:] = idx

    pick = jnp.sum(jnp.where(iota64 == idx, llm_probs, 0.0), axis=1,
                   keepdims=True)
    logp_ref[:] = jnp.log(pick)


def kernel(enhanced_posts_embeddings, selected_reasoning_embeddings,
           llm_embeddings, gate_W, gate_b, U_W, U_b, V_W, V_b, rand_u):
    uwg = jnp.concatenate([U_W.reshape(R * H, 2 * D).T, gate_W.T], axis=1)
    ubg = jnp.concatenate([U_b.reshape(1, R * H), gate_b.reshape(1, R)], axis=1)
    vwt = V_W.reshape(R * H, D).T
    vb = V_b.reshape(1, R * H)

    v_norm = pl.pallas_call(
        _v_kernel,
        out_shape=jax.ShapeDtypeStruct((NL, R * H), jnp.float32),
    )(llm_embeddings, vwt, vb)

    blk = lambda *shape: pl.BlockSpec(shape, lambda i: (0,) * len(shape))
    sel, logp, psum, msum = pl.pallas_call(
        _main_kernel,
        grid=(NBLK,),
        in_specs=[
            pl.BlockSpec((BLK, D), lambda i: (i, 0)),
            pl.BlockSpec((BLK, D), lambda i: (i, 0)),
            blk(2 * D, R * H + R),
            blk(1, R * H + R),
            blk(NL, R * H),
            pl.BlockSpec((BLK, 1), lambda i: (i, 0)),
        ],
        out_specs=[
            pl.BlockSpec((BLK, 1), lambda i: (i, 0)),
            pl.BlockSpec((BLK, 1), lambda i: (i, 0)),
            pl.BlockSpec((1, 1, R), lambda i: (i, 0, 0)),
            pl.BlockSpec((1, 1, R), lambda i: (i, 0, 0)),
        ],
        out_shape=[
            jax.ShapeDtypeStruct((B, 1), jnp.int32),
            jax.ShapeDtypeStruct((B, 1), jnp.float32),
            jax.ShapeDtypeStruct((NBLK, 1, R), jnp.float32),
            jax.ShapeDtypeStruct((NBLK, 1, R), jnp.float32),
        ],
        compiler_params=pltpu.CompilerParams(
            dimension_semantics=("parallel",),
        ),
    )(enhanced_posts_embeddings, selected_reasoning_embeddings, uwg, ubg,
      v_norm, rand_u)

    aux = pl.pallas_call(
        _aux_kernel,
        out_shape=jax.ShapeDtypeStruct((1, 1), jnp.float32),
    )(psum.reshape(NBLK, R), msum.reshape(NBLK, R))

    return sel.reshape(B), logp, aux.reshape(())


# speedup vs baseline: 1.0466x; 1.0466x over previous
import jax
import jax.numpy as jnp
from jax.experimental import pallas as pl
from jax.experimental.pallas import tpu as pltpu

B = 8192
D = 384
H = 64
R = 8
NL = 64
TEMP = 1.0
AUX = 0.05

BLK = 1024
NBLK = B // BLK

_HI = jax.lax.Precision.HIGHEST


def _v_kernel(llm_ref, vwt_ref, vb_ref, out_ref):
    vm = jnp.dot(llm_ref[:], vwt_ref[:], preferred_element_type=jnp.float32,
                 precision=_HI) + vb_ref[:]
    for r in range(R):
        sl = slice(r * H, (r + 1) * H)
        vr = vm[:, sl]
        n = jnp.sqrt(jnp.sum(vr * vr, axis=1, keepdims=True))
        out_ref[:, sl] = vr / jnp.maximum(n, 1e-12)


def _aux_kernel(psum_ref, msum_ref, aux_ref):
    p = jnp.sum(psum_ref[:], axis=0, keepdims=True)
    m = jnp.sum(msum_ref[:], axis=0, keepdims=True)
    aux_ref[:] = jnp.sum(p * m, axis=(0, 1), keepdims=True) * (R * AUX / (B * B))


def _main_kernel(xp_ref, xr_ref, uwg_ref, ubg_ref, v_ref,
                 rand_ref, sel_ref, logp_ref, psum_ref, msum_ref):
    xp = xp_ref[:]
    xr = xr_ref[:]
    uwg = uwg_ref[:]

    big = (jnp.dot(xp, uwg[:D], preferred_element_type=jnp.float32, precision=_HI)
           + jnp.dot(xr, uwg[D:], preferred_element_type=jnp.float32, precision=_HI)
           + ubg_ref[:])
    u_all = big[:, :R * H]
    logits = big[:, R * H:]

    m = jnp.max(logits, axis=1, keepdims=True)
    e = jnp.exp(logits - m)
    probs = e / jnp.sum(e, axis=1, keepdims=True)

    iota8 = jax.lax.broadcasted_iota(jnp.int32, (BLK, R), 1)
    m1 = jnp.max(logits, axis=1, keepdims=True)
    i1 = jnp.min(jnp.where(logits == m1, iota8, R), axis=1, keepdims=True)
    neg = jnp.where(iota8 == i1, -jnp.inf, logits)
    m2 = jnp.max(neg, axis=1, keepdims=True)
    i2 = jnp.min(jnp.where(neg == m2, iota8, R), axis=1, keepdims=True)

    ew = jnp.exp(m2 - m1)
    denom = 1.0 + ew
    w1 = 1.0 / denom
    w2 = ew / denom
    maskvec = (jnp.where(iota8 == i1, 1.0, 0.0) + jnp.where(iota8 == i2, 1.0, 0.0))

    psum_ref[:] = jnp.sum(probs, axis=0, keepdims=True).reshape(1, 1, R)
    msum_ref[:] = jnp.sum(maskvec, axis=0, keepdims=True).reshape(1, 1, R)

    chunkid = jax.lax.broadcasted_iota(jnp.int32, (BLK, R * H), 1) // H
    v = v_ref[:]
    llm_probs = jnp.zeros((BLK, NL), dtype=jnp.float32)
    for ik, wk in ((i1, w1), (i2, w2)):
        um = jnp.where(chunkid == ik, u_all, 0.0)
        nk = jnp.sqrt(jnp.sum(um * um, axis=1, keepdims=True))
        sck = jax.lax.dot_general(um, v, (((1,), (1,)), ((), ())),
                                  preferred_element_type=jnp.float32,
                                  precision=_HI) / jnp.maximum(nk, 1e-12)
        ms = jnp.max(sck, axis=1, keepdims=True)
        es = jnp.exp(sck - ms)
        ro = es / jnp.sum(es, axis=1, keepdims=True)
        llm_probs += wk * ro

    io_r = jax.lax.broadcasted_iota(jnp.int32, (NL, NL), 0)
    io_c = jax.lax.broadcasted_iota(jnp.int32, (NL, NL), 1)
    tri = (io_r <= io_c).astype(jnp.float32)
    csum = jnp.dot(llm_probs, tri, preferred_element_type=jnp.float32,
                   precision=_HI)
    iota64 = jax.lax.broadcasted_iota(jnp.int32, (BLK, NL), 1)
    gt = csum > rand_ref[:]
    idx = jnp.min(jnp.where(gt, iota64, NL), axis=1, keepdims=True)
    idx = jnp.where(idx == NL, 0, idx)
    sel_ref[:] = idx

    pick = jnp.sum(jnp.where(iota64 == idx, llm_probs, 0.0), axis=1,
                   keepdims=True)
    logp_ref[:] = jnp.log(pick)


def kernel(enhanced_posts_embeddings, selected_reasoning_embeddings,
           llm_embeddings, gate_W, gate_b, U_W, U_b, V_W, V_b, rand_u):
    uwg = jnp.concatenate([U_W.reshape(R * H, 2 * D).T, gate_W.T], axis=1)
    ubg = jnp.concatenate([U_b.reshape(1, R * H), gate_b.reshape(1, R)], axis=1)
    vwt = V_W.reshape(R * H, D).T
    vb = V_b.reshape(1, R * H)

    v_norm = pl.pallas_call(
        _v_kernel,
        out_shape=jax.ShapeDtypeStruct((NL, R * H), jnp.float32),
    )(llm_embeddings, vwt, vb)

    blk = lambda *shape: pl.BlockSpec(shape, lambda i: (0,) * len(shape))
    sel, logp, psum, msum = pl.pallas_call(
        _main_kernel,
        grid=(NBLK,),
        in_specs=[
            pl.BlockSpec((BLK, D), lambda i: (i, 0)),
            pl.BlockSpec((BLK, D), lambda i: (i, 0)),
            blk(2 * D, R * H + R),
            blk(1, R * H + R),
            blk(NL, R * H),
            pl.BlockSpec((BLK, 1), lambda i: (i, 0)),
        ],
        out_specs=[
            pl.BlockSpec((BLK, 1), lambda i: (i, 0)),
            pl.BlockSpec((BLK, 1), lambda i: (i, 0)),
            pl.BlockSpec((1, 1, R), lambda i: (i, 0, 0)),
            pl.BlockSpec((1, 1, R), lambda i: (i, 0, 0)),
        ],
        out_shape=[
            jax.ShapeDtypeStruct((B, 1), jnp.int32),
            jax.ShapeDtypeStruct((B, 1), jnp.float32),
            jax.ShapeDtypeStruct((NBLK, 1, R), jnp.float32),
            jax.ShapeDtypeStruct((NBLK, 1, R), jnp.float32),
        ],
        compiler_params=pltpu.CompilerParams(
            dimension_semantics=("parallel",),
        ),
    )(enhanced_posts_embeddings, selected_reasoning_embeddings, uwg, ubg,
      v_norm, rand_u)

    aux = pl.pallas_call(
        _aux_kernel,
        out_shape=jax.ShapeDtypeStruct((1, 1), jnp.float32),
    )(psum.reshape(NBLK, R), msum.reshape(NBLK, R))

    return sel.reshape(B), logp, aux.reshape(())


# speedup vs baseline: 1.0633x; 1.0160x over previous
import jax
import jax.numpy as jnp
from jax.experimental import pallas as pl
from jax.experimental.pallas import tpu as pltpu

B = 8192
D = 384
H = 64
R = 8
NL = 64
TEMP = 1.0
AUX = 0.05

BLK = 1024
NBLK = B // BLK

_HI = jax.lax.Precision.HIGHEST


def _fused_kernel(xp_ref, xr_ref, uwg_ref, ubg_ref, llm_ref, vwt_ref, vb_ref,
                  rand_ref, sel_ref, logp_ref, aux_ref, v_s, psum_s, msum_s):
    i = pl.program_id(0)

    @pl.when(i == 0)
    def _():
        vm = jnp.dot(llm_ref[:], vwt_ref[:], preferred_element_type=jnp.float32,
                     precision=_HI) + vb_ref[:]
        for r in range(R):
            sl = slice(r * H, (r + 1) * H)
            vr = vm[:, sl]
            n = jnp.sqrt(jnp.sum(vr * vr, axis=1, keepdims=True))
            v_s[:, sl] = vr / jnp.maximum(n, 1e-12)
        psum_s[:] = jnp.zeros_like(psum_s)
        msum_s[:] = jnp.zeros_like(msum_s)

    xp = xp_ref[:]
    xr = xr_ref[:]
    uwg = uwg_ref[:]

    big = (jnp.dot(xp, uwg[:D], preferred_element_type=jnp.float32, precision=_HI)
           + jnp.dot(xr, uwg[D:], preferred_element_type=jnp.float32, precision=_HI)
           + ubg_ref[:])
    u_all = big[:, :R * H]
    logits = big[:, R * H:]

    m = jnp.max(logits, axis=1, keepdims=True)
    e = jnp.exp(logits - m)
    probs = e / jnp.sum(e, axis=1, keepdims=True)

    iota8 = jax.lax.broadcasted_iota(jnp.int32, (BLK, R), 1)
    m1 = jnp.max(logits, axis=1, keepdims=True)
    i1 = jnp.min(jnp.where(logits == m1, iota8, R), axis=1, keepdims=True)
    neg = jnp.where(iota8 == i1, -jnp.inf, logits)
    m2 = jnp.max(neg, axis=1, keepdims=True)
    i2 = jnp.min(jnp.where(neg == m2, iota8, R), axis=1, keepdims=True)

    ew = jnp.exp(m2 - m1)
    denom = 1.0 + ew
    w1 = 1.0 / denom
    w2 = ew / denom
    maskvec = (jnp.where(iota8 == i1, 1.0, 0.0) + jnp.where(iota8 == i2, 1.0, 0.0))

    psum_s[:] += jnp.sum(probs, axis=0, keepdims=True)
    msum_s[:] += jnp.sum(maskvec, axis=0, keepdims=True)

    chunkid = jax.lax.broadcasted_iota(jnp.int32, (BLK, R * H), 1) // H
    v = v_s[:]
    llm_probs = jnp.zeros((BLK, NL), dtype=jnp.float32)
    for ik, wk in ((i1, w1), (i2, w2)):
        um = jnp.where(chunkid == ik, u_all, 0.0)
        nk = jnp.sqrt(jnp.sum(um * um, axis=1, keepdims=True))
        sck = jax.lax.dot_general(um, v, (((1,), (1,)), ((), ())),
                                  preferred_element_type=jnp.float32,
                                  precision=_HI) / jnp.maximum(nk, 1e-12)
        ms = jnp.max(sck, axis=1, keepdims=True)
        es = jnp.exp(sck - ms)
        ro = es / jnp.sum(es, axis=1, keepdims=True)
        llm_probs += wk * ro

    io_r = jax.lax.broadcasted_iota(jnp.int32, (NL, NL), 0)
    io_c = jax.lax.broadcasted_iota(jnp.int32, (NL, NL), 1)
    tri = (io_r <= io_c).astype(jnp.float32)
    csum = jnp.dot(llm_probs, tri, preferred_element_type=jnp.float32,
                   precision=_HI)
    iota64 = jax.lax.broadcasted_iota(jnp.int32, (BLK, NL), 1)
    gt = csum > rand_ref[:]
    idx = jnp.min(jnp.where(gt, iota64, NL), axis=1, keepdims=True)
    idx = jnp.where(idx == NL, 0, idx)
    sel_ref[:] = idx

    pick = jnp.sum(jnp.where(iota64 == idx, llm_probs, 0.0), axis=1,
                   keepdims=True)
    logp_ref[:] = jnp.log(pick)

    @pl.when(i == NBLK - 1)
    def _():
        aux_ref[:] = (jnp.sum(psum_s[:] * msum_s[:], axis=(0, 1), keepdims=True)
                      * (R * AUX / (B * B)))


def kernel(enhanced_posts_embeddings, selected_reasoning_embeddings,
           llm_embeddings, gate_W, gate_b, U_W, U_b, V_W, V_b, rand_u):
    uwg = jnp.concatenate([U_W.reshape(R * H, 2 * D).T, gate_W.T], axis=1)
    ubg = jnp.concatenate([U_b.reshape(1, R * H), gate_b.reshape(1, R)], axis=1)
    vwt = V_W.reshape(R * H, D).T
    vb = V_b.reshape(1, R * H)

    blk = lambda *shape: pl.BlockSpec(shape, lambda i: (0,) * len(shape))
    sel, logp, aux = pl.pallas_call(
        _fused_kernel,
        grid=(NBLK,),
        in_specs=[
            pl.BlockSpec((BLK, D), lambda i: (i, 0)),
            pl.BlockSpec((BLK, D), lambda i: (i, 0)),
            blk(2 * D, R * H + R),
            blk(1, R * H + R),
            blk(NL, D),
            blk(D, R * H),
            blk(1, R * H),
            pl.BlockSpec((BLK, 1), lambda i: (i, 0)),
        ],
        out_specs=[
            pl.BlockSpec((BLK, 1), lambda i: (i, 0)),
            pl.BlockSpec((BLK, 1), lambda i: (i, 0)),
            blk(1, 1),
        ],
        out_shape=[
            jax.ShapeDtypeStruct((B, 1), jnp.int32),
            jax.ShapeDtypeStruct((B, 1), jnp.float32),
            jax.ShapeDtypeStruct((1, 1), jnp.float32),
        ],
        scratch_shapes=[
            pltpu.VMEM((NL, R * H), jnp.float32),
            pltpu.VMEM((1, R), jnp.float32),
            pltpu.VMEM((1, R), jnp.float32),
        ],
        compiler_params=pltpu.CompilerParams(
            dimension_semantics=("arbitrary",),
        ),
    )(enhanced_posts_embeddings, selected_reasoning_embeddings, uwg, ubg,
      llm_embeddings, vwt, vb, rand_u)

    return sel.reshape(B), logp, aux.reshape(())


# speedup vs baseline: 1.1157x; 1.0493x over previous
import jax
import jax.numpy as jnp
from jax.experimental import pallas as pl
from jax.experimental.pallas import tpu as pltpu

B = 8192
D = 384
H = 64
R = 8
NL = 64
TEMP = 1.0
AUX = 0.05

BLK = 1024
NBLK = B // BLK

_HI = jax.lax.Precision.HIGHEST


def _nt(a, b):
    return jax.lax.dot_general(a, b, (((1,), (1,)), ((), ())),
                               preferred_element_type=jnp.float32,
                               precision=_HI)


def _fused_kernel(xp_ref, xr_ref, uw_ref, ub_ref, gw_ref, gb_ref, llm_ref,
                  vw_ref, vb_ref, rand_ref, sel_ref, logp_ref, aux_ref,
                  v_s, psum_s, msum_s):
    i = pl.program_id(0)

    @pl.when(i == 0)
    def _():
        vm = _nt(llm_ref[:], vw_ref[:]) + vb_ref[:]
        for r in range(R):
            sl = slice(r * H, (r + 1) * H)
            vr = vm[:, sl]
            n = jnp.sqrt(jnp.sum(vr * vr, axis=1, keepdims=True))
            v_s[:, sl] = vr / jnp.maximum(n, 1e-12)
        psum_s[:] = jnp.zeros_like(psum_s)
        msum_s[:] = jnp.zeros_like(msum_s)

    xp = xp_ref[:]
    xr = xr_ref[:]
    uw = uw_ref[:]
    gw = gw_ref[:]

    u_all = (_nt(xp, uw[:, :D]) + _nt(xr, uw[:, D:]) + ub_ref[:])
    logits = (_nt(xp, gw[:, :D]) + _nt(xr, gw[:, D:]) + gb_ref[:])

    m = jnp.max(logits, axis=1, keepdims=True)
    e = jnp.exp(logits - m)
    probs = e / jnp.sum(e, axis=1, keepdims=True)

    iota8 = jax.lax.broadcasted_iota(jnp.int32, (BLK, R), 1)
    m1 = jnp.max(logits, axis=1, keepdims=True)
    i1 = jnp.min(jnp.where(logits == m1, iota8, R), axis=1, keepdims=True)
    neg = jnp.where(iota8 == i1, -jnp.inf, logits)
    m2 = jnp.max(neg, axis=1, keepdims=True)
    i2 = jnp.min(jnp.where(neg == m2, iota8, R), axis=1, keepdims=True)

    ew = jnp.exp(m2 - m1)
    denom = 1.0 + ew
    w1 = 1.0 / denom
    w2 = ew / denom
    maskvec = (jnp.where(iota8 == i1, 1.0, 0.0) + jnp.where(iota8 == i2, 1.0, 0.0))

    psum_s[:] += jnp.sum(probs, axis=0, keepdims=True)
    msum_s[:] += jnp.sum(maskvec, axis=0, keepdims=True)

    chunkid = jax.lax.broadcasted_iota(jnp.int32, (BLK, R * H), 1) // H
    v = v_s[:]
    llm_probs = jnp.zeros((BLK, NL), dtype=jnp.float32)
    for ik, wk in ((i1, w1), (i2, w2)):
        um = jnp.where(chunkid == ik, u_all, 0.0)
        nk = jnp.sqrt(jnp.sum(um * um, axis=1, keepdims=True))
        sck = _nt(um, v) / jnp.maximum(nk, 1e-12)
        ms = jnp.max(sck, axis=1, keepdims=True)
        es = jnp.exp(sck - ms)
        ro = es / jnp.sum(es, axis=1, keepdims=True)
        llm_probs += wk * ro

    io_r = jax.lax.broadcasted_iota(jnp.int32, (NL, NL), 0)
    io_c = jax.lax.broadcasted_iota(jnp.int32, (NL, NL), 1)
    tri = (io_r <= io_c).astype(jnp.float32)
    csum = jnp.dot(llm_probs, tri, preferred_element_type=jnp.float32,
                   precision=_HI)
    iota64 = jax.lax.broadcasted_iota(jnp.int32, (BLK, NL), 1)
    gt = csum > rand_ref[:]
    idx = jnp.min(jnp.where(gt, iota64, NL), axis=1, keepdims=True)
    idx = jnp.where(idx == NL, 0, idx)
    sel_ref[:] = idx

    pick = jnp.sum(jnp.where(iota64 == idx, llm_probs, 0.0), axis=1,
                   keepdims=True)
    logp_ref[:] = jnp.log(pick)

    @pl.when(i == NBLK - 1)
    def _():
        aux_ref[:] = (jnp.sum(psum_s[:] * msum_s[:], axis=(0, 1), keepdims=True)
                      * (R * AUX / (B * B)))


def kernel(enhanced_posts_embeddings, selected_reasoning_embeddings,
           llm_embeddings, gate_W, gate_b, U_W, U_b, V_W, V_b, rand_u):
    uw = U_W.reshape(R * H, 2 * D)
    ub = U_b.reshape(1, R * H)
    gb = gate_b.reshape(1, R)
    vw = V_W.reshape(R * H, D)
    vb = V_b.reshape(1, R * H)

    blk = lambda *shape: pl.BlockSpec(shape, lambda i: (0,) * len(shape))
    sel, logp, aux = pl.pallas_call(
        _fused_kernel,
        grid=(NBLK,),
        in_specs=[
            pl.BlockSpec((BLK, D), lambda i: (i, 0)),
            pl.BlockSpec((BLK, D), lambda i: (i, 0)),
            blk(R * H, 2 * D),
            blk(1, R * H),
            blk(R, 2 * D),
            blk(1, R),
            blk(NL, D),
            blk(R * H, D),
            blk(1, R * H),
            pl.BlockSpec((BLK, 1), lambda i: (i, 0)),
        ],
        out_specs=[
            pl.BlockSpec((BLK, 1), lambda i: (i, 0)),
            pl.BlockSpec((BLK, 1), lambda i: (i, 0)),
            blk(1, 1),
        ],
        out_shape=[
            jax.ShapeDtypeStruct((B, 1), jnp.int32),
            jax.ShapeDtypeStruct((B, 1), jnp.float32),
            jax.ShapeDtypeStruct((1, 1), jnp.float32),
        ],
        scratch_shapes=[
            pltpu.VMEM((NL, R * H), jnp.float32),
            pltpu.VMEM((1, R), jnp.float32),
            pltpu.VMEM((1, R), jnp.float32),
        ],
        compiler_params=pltpu.CompilerParams(
            dimension_semantics=("arbitrary",),
        ),
    )(enhanced_posts_embeddings, selected_reasoning_embeddings, uw, ub,
      gate_W, gb, llm_embeddings, vw, vb, rand_u)

    return sel.reshape(B), logp, aux.reshape(())
